# baseline (speedup 1.0000x reference)
import functools

import jax
import jax.numpy as jnp
from jax import lax
from jax.experimental import pallas as pl
from jax.experimental.pallas import tpu as pltpu
from jax.experimental.pallas import tpu_sc as plsc

N = 10000
D = 128
E = 320000
NC = 2
NS = 16
NW = NC * NS
EDGES_PER_W = E // NW
CHUNK = 80
NCHUNKS = EDGES_PER_W // CHUNK
N_PAD = 10240
ROWS_PER_TILE = N_PAD // NS
ZROWS = 16


def _sc_layer(h, src3d, dst1d):
    mesh = plsc.VectorSubcoreMesh(core_axis_name="c", subcore_axis_name="s")
    NBUF = 3

    @functools.partial(
        pl.kernel,
        out_type=jax.ShapeDtypeStruct((NC, N_PAD, D), jnp.float32),
        mesh=mesh,
        scratch_types=(
            [pltpu.VMEM((CHUNK,), jnp.int32) for _ in range(NBUF)]
            + [pltpu.VMEM((CHUNK,), jnp.int32) for _ in range(NBUF)]
            + [pltpu.VMEM((CHUNK, D), jnp.float32) for _ in range(NBUF)]
            + [pltpu.VMEM((ZROWS, D), jnp.float32)]
            + [pltpu.VMEM_SHARED((N_PAD, D), jnp.float32)]
            + [pltpu.SemaphoreType.DMA for _ in range(3 * NBUF)]
        ),
    )
    def layer_kernel(h_hbm, src_hbm, dst_hbm, out_hbm, *refs):
        src_v = refs[0:NBUF]
        dst_v = refs[NBUF:2 * NBUF]
        rows_v = refs[2 * NBUF:3 * NBUF]
        zbuf = refs[3 * NBUF]
        acc_sh = refs[3 * NBUF + 1]
        sem_s = refs[3 * NBUF + 2:3 * NBUF + 2 + NBUF]
        sem_d = refs[3 * NBUF + 2 + NBUF:3 * NBUF + 2 + 2 * NBUF]
        sem_g = refs[3 * NBUF + 2 + 2 * NBUF:3 * NBUF + 2 + 3 * NBUF]

        c = lax.axis_index("c")
        s = lax.axis_index("s")
        wid = s * NC + c
        ebase = wid * EDGES_PER_W

        def start_idx(j, b):
            off = ebase + j * CHUNK
            pltpu.async_copy(src_hbm.at[pl.ds(off, CHUNK)], src_v[b],
                             sem_s[b])
            pltpu.async_copy(dst_hbm.at[pl.ds(off, CHUNK)], dst_v[b],
                             sem_d[b])

        def wait_src(j, b):
            off = ebase + j * CHUNK
            pltpu.make_async_copy(src_hbm.at[pl.ds(off, CHUNK)], src_v[b],
                                  sem_s[b]).wait()

        def wait_dst(j, b):
            off = ebase + j * CHUNK
            pltpu.make_async_copy(dst_hbm.at[pl.ds(off, CHUNK)], dst_v[b],
                                  sem_d[b]).wait()

        def start_gather(b):
            pltpu.async_copy(h_hbm.at[src_v[b]], rows_v[b], sem_g[b])

        def wait_gather(b):
            pltpu.make_async_copy(h_hbm.at[src_v[b]], rows_v[b],
                                  sem_g[b]).wait()

        def scatter_add(b):
            pltpu.sync_copy(rows_v[b], acc_sh.at[dst_v[b]], add=True)

        start_idx(0, 0)
        start_idx(1, 1)
        start_idx(2, 2)

        @pl.loop(0, ZROWS)
        def _(r):
            @pl.loop(0, D, step=16)
            def _(j):
                zbuf.at[r, pl.ds(j, 16)][...] = jnp.zeros((16,), jnp.float32)

        row0 = s * ROWS_PER_TILE

        @pl.loop(0, ROWS_PER_TILE // ZROWS)
        def _(k):
            pltpu.sync_copy(zbuf, acc_sh.at[pl.ds(row0 + k * ZROWS, ZROWS)])

        wait_src(0, 0)
        start_gather(0)
        wait_src(1, 1)
        start_gather(1)

        plsc.subcore_barrier()

        def step(m, b, do_g2, do_i3):
            if do_g2:
                wait_src(m + 2, (b + 2) % NBUF)
                start_gather((b + 2) % NBUF)
            wait_gather(b)
            wait_dst(m, b)
            scatter_add(b)
            if do_i3:
                start_idx(m + 3, b)

        LOOPED = (NCHUNKS - 5) // NBUF * NBUF

        @pl.loop(0, LOOPED, step=NBUF)
        def _(m):
            step(m, 0, True, True)
            step(m + 1, 1, True, True)
            step(m + 2, 2, True, True)

        for m in range(LOOPED, NCHUNKS):
            step(m, m % NBUF, m + 2 < NCHUNKS, m + 3 < NCHUNKS)

        plsc.subcore_barrier()

        pltpu.sync_copy(acc_sh.at[pl.ds(row0, ROWS_PER_TILE)],
                        out_hbm.at[c].at[pl.ds(row0, ROWS_PER_TILE)])

    return layer_kernel(h, src3d, dst1d)


def _sum_partials(p):
    def body(p_ref, o_ref):
        o_ref[...] = p_ref[0] + p_ref[1]

    return pl.pallas_call(
        body,
        out_shape=jax.ShapeDtypeStruct((N, D), jnp.float32),
        grid=(10,),
        in_specs=[pl.BlockSpec((NC, N // 10, D), lambda i: (0, i, 0))],
        out_specs=pl.BlockSpec((N // 10, D), lambda i: (i, 0)),
    )(p)


def _head(p, W1, b1, W2, b2):
    def body(p_ref, w1_ref, b1_ref, w2_ref, b2_ref, o_ref):
        h = p_ref[0] + p_ref[1]
        h = jnp.dot(h, w1_ref[...], preferred_element_type=jnp.float32)
        h = jnp.maximum(h + b1_ref[...], 0.0)
        o = jnp.sum(h * w2_ref[...], axis=1, keepdims=True)
        o_ref[...] = o + b2_ref[0]

    return pl.pallas_call(
        body,
        out_shape=jax.ShapeDtypeStruct((N, 1), jnp.float32),
        grid=(1,),
        in_specs=[
            pl.BlockSpec((NC, N, D), lambda i: (0, 0, 0)),
            pl.BlockSpec((D, D), lambda i: (0, 0)),
            pl.BlockSpec((1, D), lambda i: (0, 0)),
            pl.BlockSpec((1, D), lambda i: (0, 0)),
            pl.BlockSpec(memory_space=pltpu.SMEM),
        ],
        out_specs=pl.BlockSpec((N, 1), lambda i: (0, 0)),
    )(p, W1, b1.reshape(1, D), W2.reshape(1, D), b2)


def kernel(x, edge_index, W1, b1, W2, b2):
    src = edge_index[0]
    dst = edge_index[1]
    p1 = _sc_layer(x, src, dst)
    h1 = _sum_partials(p1)
    p2 = _sc_layer(h1, src, dst)
    return _head(p2, W1, b1, W2, b2)
